# baseline (speedup 1.0000x reference)
import functools

import jax
import jax.numpy as jnp
from jax import lax
from jax.experimental import pallas as pl
from jax.experimental.pallas import tpu as pltpu
from jax.experimental.pallas import tpu_sc as plsc

_LANES = 16


def _pe_add_call(x, enc_weight, cls_row):
    B, T, D = x.shape
    T1 = T + 1
    dtype = x.dtype

    mesh = plsc.VectorSubcoreMesh(core_axis_name="c", subcore_axis_name="s")
    num_workers = mesh.num_cores * mesh.num_subcores
    assert T % num_workers == 0
    rows_per_worker = T // num_workers
    chunk = 16
    assert rows_per_worker % chunk == 0
    n_chunks = rows_per_worker // chunk
    n_vecs = D // _LANES

    @functools.partial(
        pl.kernel,
        out_type=jax.ShapeDtypeStruct((B, T1, D), dtype),
        mesh=mesh,
        scratch_types=[
            pltpu.VMEM((chunk, D), dtype),
            pltpu.VMEM((chunk, D), dtype),
            pltpu.VMEM((1, D), dtype),
            pltpu.VMEM((1, D), dtype),
        ],
        compiler_params=pltpu.CompilerParams(use_tc_tiling_on_sc=False),
    )
    def pe_add(x_hbm, pe_hbm, cls_hbm, out_hbm, x_v, pe_v, cls_v, pe0_v):
        wid = lax.axis_index("s") * mesh.num_cores + lax.axis_index("c")
        base = wid * rows_per_worker

        @pl.when(wid == 0)
        def _():
            pltpu.sync_copy(cls_hbm, cls_v)
            pltpu.sync_copy(pe_hbm.at[pl.ds(0, 1)], pe0_v)
            for j in range(n_vecs):
                sl = pl.ds(j * _LANES, _LANES)
                cls_v[0, sl] = cls_v[0, sl] + pe0_v[0, sl]
            for b in range(B):
                pltpu.sync_copy(cls_v, out_hbm.at[b, pl.ds(0, 1)])

        for c in range(n_chunks):
            r = base + c * chunk
            pltpu.sync_copy(pe_hbm.at[pl.ds(r + 1, chunk)], pe_v)
            for b in range(B):
                pltpu.sync_copy(x_hbm.at[b, pl.ds(r, chunk)], x_v)

                def row_add(i, carry):
                    for j in range(n_vecs):
                        sl = pl.ds(j * _LANES, _LANES)
                        x_v[i, sl] = x_v[i, sl] + pe_v[i, sl]
                    return carry

                lax.fori_loop(0, chunk, row_add, 0)
                pltpu.sync_copy(x_v, out_hbm.at[b, pl.ds(r + 1, chunk)])

    return pe_add(x, enc_weight, cls_row)


def kernel(x, enc_weight, cls_tokens_stream, cls_tokens_view, is_stream,
           stream_id, is_view, view_id, use_cls):
    B, T, D = x.shape
    cls_stream = lax.dynamic_slice_in_dim(cls_tokens_stream, stream_id, 1, axis=0)
    cls_view = lax.dynamic_slice_in_dim(cls_tokens_view, view_id, 1, axis=0)
    cls_zero = jnp.zeros((1, 1, D), dtype=x.dtype)
    cls_tok = jnp.where(
        jnp.asarray(is_stream) != 0,
        cls_stream,
        jnp.where(jnp.asarray(is_view) != 0, cls_view, cls_zero),
    )
    cls_tok = cls_tok * jnp.asarray(use_cls, dtype=x.dtype)
    cls_row = cls_tok.reshape(1, D)
    return _pe_add_call(x, enc_weight, cls_row)


# speedup vs baseline: 1.1920x; 1.1920x over previous
import functools

import jax
import jax.numpy as jnp
from jax import lax
from jax.experimental import pallas as pl
from jax.experimental.pallas import tpu as pltpu
from jax.experimental.pallas import tpu_sc as plsc

_LANES = 16


def _pe_add_call(x, enc_weight, cls_row):
    B, T, D = x.shape
    T1 = T + 1
    dtype = x.dtype

    mesh = plsc.VectorSubcoreMesh(core_axis_name="c", subcore_axis_name="s")
    num_workers = mesh.num_cores * mesh.num_subcores
    assert T % num_workers == 0
    rows_per_worker = T // num_workers
    chunk = 16
    assert rows_per_worker % chunk == 0
    n_chunks = rows_per_worker // chunk
    n_vecs = D // _LANES

    nbuf = 3
    n_steps = n_chunks * B

    @functools.partial(
        pl.kernel,
        out_type=jax.ShapeDtypeStruct((B, T1, D), dtype),
        mesh=mesh,
        scratch_types=[
            pltpu.VMEM((chunk, D), dtype),
            pltpu.VMEM((chunk, D), dtype),
            pltpu.VMEM((chunk, D), dtype),
            pltpu.VMEM((chunk, D), dtype),
            pltpu.VMEM((chunk, D), dtype),
            pltpu.VMEM((1, D), dtype),
            pltpu.VMEM((1, D), dtype),
            pltpu.SemaphoreType.DMA,
            pltpu.SemaphoreType.DMA,
            pltpu.SemaphoreType.DMA,
            pltpu.SemaphoreType.DMA,
            pltpu.SemaphoreType.DMA,
            pltpu.SemaphoreType.DMA,
            pltpu.SemaphoreType.DMA,
            pltpu.SemaphoreType.DMA,
        ],
        compiler_params=pltpu.CompilerParams(use_tc_tiling_on_sc=False),
    )
    def pe_add(x_hbm, pe_hbm, cls_hbm, out_hbm,
               xb0, xb1, xb2, peb0, peb1, cls_v, pe0_v,
               sx0, sx1, sx2, so0, so1, so2, sp0, sp1):
        xb = [xb0, xb1, xb2]
        peb = [peb0, peb1]
        sx = [sx0, sx1, sx2]
        so = [so0, so1, so2]
        sp = [sp0, sp1]
        wid = lax.axis_index("s") * mesh.num_cores + lax.axis_index("c")
        base = wid * rows_per_worker

        @pl.when(wid == 0)
        def _():
            pltpu.sync_copy(cls_hbm, cls_v)
            pltpu.sync_copy(pe_hbm.at[pl.ds(0, 1)], pe0_v)
            for j in range(n_vecs):
                sl = pl.ds(j * _LANES, _LANES)
                cls_v[0, sl] = cls_v[0, sl] + pe0_v[0, sl]
            for b in range(B):
                pltpu.sync_copy(cls_v, out_hbm.at[b, pl.ds(0, 1)])

        pe_d = [None] * n_chunks
        x_d = [None] * n_steps
        out_d = [None] * n_steps

        def start_x(s):
            c, b = s // B, s % B
            return pltpu.async_copy(
                x_hbm.at[b, pl.ds(base + c * chunk, chunk)], xb[s % nbuf],
                sx[s % nbuf])

        pe_d[0] = pltpu.async_copy(pe_hbm.at[pl.ds(base + 1, chunk)], peb[0], sp[0])
        if n_chunks > 1:
            pe_d[1] = pltpu.async_copy(
                pe_hbm.at[pl.ds(base + chunk + 1, chunk)], peb[1], sp[1])
        x_d[0] = start_x(0)

        for s in range(n_steps):
            c, b = s // B, s % B
            if s + 1 < n_steps:
                if s - (nbuf - 1) >= 0:
                    out_d[s - (nbuf - 1)].wait()
                x_d[s + 1] = start_x(s + 1)
            if b == 0:
                pe_d[c].wait()
                if c >= 1 and c + 1 < n_chunks:
                    pe_d[c + 1] = pltpu.async_copy(
                        pe_hbm.at[pl.ds(base + (c + 1) * chunk + 1, chunk)],
                        peb[(c + 1) % 2], sp[(c + 1) % 2])
            x_d[s].wait()
            xv, pv = xb[s % nbuf], peb[c % 2]

            def row_add(i, carry):
                for j in range(n_vecs):
                    sl = pl.ds(j * _LANES, _LANES)
                    xv[i, sl] = xv[i, sl] + pv[i, sl]
                return carry

            lax.fori_loop(0, chunk, row_add, 0)
            out_d[s] = pltpu.async_copy(
                xv, out_hbm.at[b, pl.ds(base + c * chunk + 1, chunk)],
                so[s % nbuf])

        for s in range(max(0, n_steps - nbuf), n_steps):
            out_d[s].wait()

    return pe_add(x, enc_weight, cls_row)


def kernel(x, enc_weight, cls_tokens_stream, cls_tokens_view, is_stream,
           stream_id, is_view, view_id, use_cls):
    B, T, D = x.shape
    cls_stream = lax.dynamic_slice_in_dim(cls_tokens_stream, stream_id, 1, axis=0)
    cls_view = lax.dynamic_slice_in_dim(cls_tokens_view, view_id, 1, axis=0)
    cls_zero = jnp.zeros((1, 1, D), dtype=x.dtype)
    cls_tok = jnp.where(
        jnp.asarray(is_stream) != 0,
        cls_stream,
        jnp.where(jnp.asarray(is_view) != 0, cls_view, cls_zero),
    )
    cls_tok = cls_tok * jnp.asarray(use_cls, dtype=x.dtype)
    cls_row = cls_tok.reshape(1, D)
    return _pe_add_call(x, enc_weight, cls_row)
